# baseline (speedup 1.0000x reference)
import jax
import jax.numpy as jnp
from jax.experimental import pallas as pl
from jax.experimental.pallas import tpu as pltpu


def _body(inp_ref, x_ref, wl_ref, wi_ref, adj_ref, bias_ref, out_ref, y_ref):
    l = pl.program_id(1)

    @pl.when(pl.program_id(0) == 0)
    def _build_y():
        y_ref[pl.ds(l, 1)] = (
            jnp.dot(inp_ref[...], wl_ref[0], preferred_element_type=jnp.float32)
            + jnp.dot(x_ref[...], wi_ref[0], preferred_element_type=jnp.float32)
        ).astype(jnp.bfloat16)[None]

    acc = jnp.dot(adj_ref[0].astype(jnp.bfloat16), y_ref[l],
                  preferred_element_type=jnp.float32)

    @pl.when(l == 0)
    def _init():
        out_ref[...] = acc + bias_ref[...]

    @pl.when(l != 0)
    def _accum():
        out_ref[...] += acc


def kernel(input, X, adj_list, W_line, W_inp, graph_mixing_weight,
           inp_graph_mixing_weight, bias):
    N, d = input.shape
    L = adj_list.shape[0]

    wl = jnp.swapaxes(W_line * graph_mixing_weight[:, :, None], 1, 2)
    wi = jnp.swapaxes(W_inp * inp_graph_mixing_weight[:, :, None], 1, 2)
    bias2d = bias.reshape(1, d)

    BM = 256
    num_m = N // BM
    out = pl.pallas_call(
        _body,
        grid=(num_m, L),
        in_specs=[
            pl.BlockSpec((N, d), lambda m, l: (0, 0)),
            pl.BlockSpec((N, d), lambda m, l: (0, 0)),
            pl.BlockSpec((1, d, d), lambda m, l: (l, 0, 0)),
            pl.BlockSpec((1, d, d), lambda m, l: (l, 0, 0)),
            pl.BlockSpec((1, BM, N), lambda m, l: (l, m, 0)),
            pl.BlockSpec((1, d), lambda m, l: (0, 0)),
        ],
        out_specs=pl.BlockSpec((BM, d), lambda m, l: (m, 0)),
        out_shape=jax.ShapeDtypeStruct((N, d), jnp.float32),
        scratch_shapes=[pltpu.VMEM((L, N, d), jnp.bfloat16)],
        compiler_params=pltpu.CompilerParams(
            dimension_semantics=("arbitrary", "arbitrary"),
        ),
    )(input, X, wl, wi, adj_list, bias2d)
    return out


# speedup vs baseline: 1.1959x; 1.1959x over previous
import jax
import jax.numpy as jnp
from jax.experimental import pallas as pl
from jax.experimental.pallas import tpu as pltpu


def _body(inp_ref, x_ref, wl_ref, wi_ref, adj_ref, bias_ref, out_ref, y_ref):
    l = pl.program_id(1)

    @pl.when(pl.program_id(0) == 0)
    def _build_y():
        y_ref[pl.ds(l, 1)] = (
            jnp.dot(inp_ref[...].astype(jnp.bfloat16), wl_ref[0],
                    preferred_element_type=jnp.float32)
            + jnp.dot(x_ref[...].astype(jnp.bfloat16), wi_ref[0],
                      preferred_element_type=jnp.float32)
        ).astype(jnp.bfloat16)[None]

    acc = jnp.dot(adj_ref[0].astype(jnp.bfloat16), y_ref[l],
                  preferred_element_type=jnp.float32)

    @pl.when(l == 0)
    def _init():
        out_ref[...] = acc + bias_ref[...]

    @pl.when(l != 0)
    def _accum():
        out_ref[...] += acc


def kernel(input, X, adj_list, W_line, W_inp, graph_mixing_weight,
           inp_graph_mixing_weight, bias):
    N, d = input.shape
    L = adj_list.shape[0]

    wl = jnp.swapaxes(W_line * graph_mixing_weight[:, :, None], 1, 2).astype(jnp.bfloat16)
    wi = jnp.swapaxes(W_inp * inp_graph_mixing_weight[:, :, None], 1, 2).astype(jnp.bfloat16)
    bias2d = bias.reshape(1, d)

    BM = 512
    num_m = N // BM
    out = pl.pallas_call(
        _body,
        grid=(num_m, L),
        in_specs=[
            pl.BlockSpec((N, d), lambda m, l: (0, 0)),
            pl.BlockSpec((N, d), lambda m, l: (0, 0)),
            pl.BlockSpec((1, d, d), lambda m, l: (l, 0, 0)),
            pl.BlockSpec((1, d, d), lambda m, l: (l, 0, 0)),
            pl.BlockSpec((1, BM, N), lambda m, l: (l, m, 0)),
            pl.BlockSpec((1, d), lambda m, l: (0, 0)),
        ],
        out_specs=pl.BlockSpec((BM, d), lambda m, l: (m, 0)),
        out_shape=jax.ShapeDtypeStruct((N, d), jnp.float32),
        scratch_shapes=[pltpu.VMEM((L, N, d), jnp.bfloat16)],
        compiler_params=pltpu.CompilerParams(
            dimension_semantics=("arbitrary", "arbitrary"),
        ),
    )(input, X, wl, wi, adj_list, bias2d)
    return out
